# Initial kernel scaffold; baseline (speedup 1.0000x reference)
import jax
import jax.numpy as jnp
from jax.experimental import pallas as pl


def kernel(all_queries_0, all_queries_1, gc_output, lc_output, attn_hard, gc_spatial_res, lc_spatial_res):
    raise NotImplementedError("write your pallas kernel here")


# speedup vs baseline: 2.2497x; 2.2497x over previous
import jax
import jax.numpy as jnp
from jax.experimental import pallas as pl
from jax.experimental.pallas import tpu as pltpu

_B, _RES, _C, _Q = 64, 28, 384, 5
_N = _RES * _RES
_NEG_PRESSURE = 0.1
_BIG_NEG = 1e30


def _inv_norm(x):
    ss = jnp.sum(x * x, axis=-1, keepdims=True)
    return 1.0 / jnp.maximum(jnp.sqrt(ss), 1e-10)


def _batch_kernel(q0_ref, q1_ref, gc_ref, lc_ref, mg_ref, ml_ref, out_ref):
    gc = gc_ref[0]
    lc = lc_ref[0]
    q0 = q0_ref[0]
    q1 = q1_ref[0]
    mg = mg_ref[0].astype(jnp.float32)
    ml = ml_ref[0].astype(jnp.float32)

    col8 = jax.lax.broadcasted_iota(jnp.int32, (_N, 8), 1)
    zpad = jnp.zeros((3, _C), jnp.float32)

    inv_l = _inv_norm(lc)
    q1n = q1 * _inv_norm(q1)
    rhs_l = jnp.concatenate([q1n, zpad], axis=0)
    la = jax.lax.dot_general(lc, rhs_l, (((1,), (1,)), ((), ())),
                             preferred_element_type=jnp.float32)
    a_l = jnp.where(col8 < _Q, jnp.maximum(la * inv_l, 0.0), -_BIG_NEG)
    e_l = jnp.exp(a_l - jnp.max(a_l, axis=1, keepdims=True))
    lc_s = e_l / jnp.sum(e_l, axis=1, keepdims=True)
    v_l = jnp.where(col8 < _Q, lc_s * ml * inv_l,
                    jnp.where(col8 == _Q, inv_l, 0.0))
    b8 = jax.lax.dot_general(v_l, lc, (((0,), (0,)), ((), ())),
                             preferred_element_type=jnp.float32)
    s_lc = jnp.sum(lc_s * ml, axis=0, keepdims=True)

    inv_g = _inv_norm(gc)
    q0n = q0 * _inv_norm(q0)
    lc_sum = b8[_Q:_Q + 1, :]
    rhs_g = jnp.concatenate([q0n, lc_sum, zpad[:2]], axis=0)
    ga = jax.lax.dot_general(gc, rhs_g, (((1,), (1,)), ((), ())),
                             preferred_element_type=jnp.float32)
    a_g = jnp.where(col8 < _Q, jnp.maximum(ga * inv_g, 0.0), -_BIG_NEG)
    e_g = jnp.exp(a_g - jnp.max(a_g, axis=1, keepdims=True))
    gc_s = e_g / jnp.sum(e_g, axis=1, keepdims=True)
    w_g = jnp.where(col8 < _Q, gc_s * mg * inv_g,
                    jnp.where(col8 == _Q, inv_g, 0.0))
    a8 = jax.lax.dot_general(w_g, gc, (((0,), (0,)), ((), ())),
                             preferred_element_type=jnp.float32)

    rowmean = ga[:, _Q:_Q + 1] * inv_g * (1.0 / _N)
    tvec = jnp.sum(gc_s * s_lc, axis=1, keepdims=True) * mg

    ab = a8 * b8
    row8 = jax.lax.broadcasted_iota(jnp.int32, (8, _C), 0)
    p1 = jnp.sum(jnp.where(row8 < _Q, ab, 0.0))
    g = jnp.sum(jnp.where(row8 == _Q, ab, 0.0))
    p2 = jnp.sum(tvec)
    p3 = jnp.sum(tvec * rowmean)

    z = jnp.concatenate([q0n, q1n], axis=0)
    sim = jax.lax.dot_general(z, z, (((1,), (1,)), ((), ())),
                              preferred_element_type=jnp.float32)
    ri = jax.lax.broadcasted_iota(jnp.int32, (2 * _Q, 2 * _Q), 0)
    ci = jax.lax.broadcasted_iota(jnp.int32, (2 * _Q, 2 * _Q), 1)
    simm = jnp.where(ri == ci, -_BIG_NEG, sim)
    m = jnp.max(simm, axis=1, keepdims=True)
    lse = jnp.log(jnp.sum(jnp.exp(simm - m), axis=1, keepdims=True)) + m
    pos = jnp.sum(jnp.where(ci == (ri + _Q) % (2 * _Q), sim, 0.0),
                  axis=1, keepdims=True)
    ce_sum = jnp.sum(lse - pos)

    lane = jax.lax.broadcasted_iota(jnp.int32, (1, 1, 128), 2)
    out_ref[...] = (jnp.where(lane == 0, p1, 0.0)
                    + jnp.where(lane == 1, p2, 0.0)
                    + jnp.where(lane == 2, p3, 0.0)
                    + jnp.where(lane == 3, g, 0.0)
                    + jnp.where(lane == 4, ce_sum, 0.0))


def _combine_kernel(p_ref, out_ref):
    p = p_ref[...]
    s = jnp.sum(p, axis=0, keepdims=True)
    lane = jax.lax.broadcasted_iota(jnp.int32, (1, 128), 1)

    def pick(i):
        return jnp.sum(jnp.where(lane == i, s, 0.0))

    s1, s2, s3, sg, sce = pick(0), pick(1), pick(2), pick(3), pick(4)
    old_mean = sg / (_B * _N * _N)
    cor_loss = -0.15 * (s1 + (old_mean - _NEG_PRESSURE) * s2 - s3)
    qa_loss = sce / (_B * 2 * _Q)
    out_ref[...] = jnp.where(lane == 0, cor_loss + qa_loss, 0.0)


def kernel(all_queries_0, all_queries_1, gc_output, lc_output,
           attn_hard, gc_spatial_res, lc_spatial_res):
    del gc_spatial_res, lc_spatial_res
    lc = lc_output[:, 0]
    attn3 = attn_hard.reshape(2 * _B, _N, 1)

    partials = pl.pallas_call(
        _batch_kernel,
        grid=(_B,),
        in_specs=[
            pl.BlockSpec((1, _Q, _C), lambda b: (b, 0, 0)),
            pl.BlockSpec((1, _Q, _C), lambda b: (b, 0, 0)),
            pl.BlockSpec((1, _N, _C), lambda b: (b, 0, 0)),
            pl.BlockSpec((1, _N, _C), lambda b: (b, 0, 0)),
            pl.BlockSpec((1, _N, 1), lambda b: (b, 0, 0)),
            pl.BlockSpec((1, _N, 1), lambda b: (b + _B, 0, 0)),
        ],
        out_specs=pl.BlockSpec((1, 1, 128), lambda b: (b, 0, 0)),
        out_shape=jax.ShapeDtypeStruct((_B, 1, 128), jnp.float32),
        compiler_params=pltpu.CompilerParams(
            dimension_semantics=("parallel",)),
    )(all_queries_0, all_queries_1, gc_output, lc, attn3, attn3)

    out = pl.pallas_call(
        _combine_kernel,
        grid=(1,),
        in_specs=[pl.BlockSpec((_B, 128), lambda i: (0, 0))],
        out_specs=pl.BlockSpec((1, 128), lambda i: (0, 0)),
        out_shape=jax.ShapeDtypeStruct((1, 128), jnp.float32),
    )(partials.reshape(_B, 128))
    return out[0, 0]


# speedup vs baseline: 4.9842x; 2.2155x over previous
import jax
import jax.numpy as jnp
from jax.experimental import pallas as pl
from jax.experimental.pallas import tpu as pltpu

_B, _RES, _C, _Q = 64, 28, 384, 5
_N = _RES * _RES
_NEG_PRESSURE = 0.1
_BIG_NEG = 1e30

_CONTRACT_C = (((1,), (1,)), ((), ()))
_CONTRACT_N = (((1,), (0,)), ((), ()))


def _row_inv_norm(x):
    ss = jnp.sum(x * x, axis=-1, keepdims=True)
    return 1.0 / jnp.maximum(jnp.sqrt(ss), 1e-10)


def _side(data, qn, extra_row, mask_row, ones_row):
    ssq = jax.lax.dot_general(ones_row, data * data, _CONTRACT_C,
                              preferred_element_type=jnp.float32)
    inv = 1.0 / jnp.maximum(jnp.sqrt(ssq), 1e-10)
    lhs = jnp.concatenate(
        [qn, extra_row, jnp.zeros((2, _C), jnp.float32)], axis=0)
    raw = jax.lax.dot_general(lhs, data, _CONTRACT_C,
                              preferred_element_type=jnp.float32)
    a = jnp.maximum(raw[:_Q] * inv, 0.0)
    e = jnp.exp(a - jnp.max(a, axis=0, keepdims=True))
    soft = e / jnp.sum(e, axis=0, keepdims=True)
    w8 = jnp.concatenate(
        [soft * mask_row * inv, inv, jnp.zeros((2, _N), jnp.float32)],
        axis=0)
    return soft, w8, inv, raw[_Q:_Q + 1]


def _batch_kernel(q0_ref, q1_ref, gc_ref, lc_ref, mg_ref, ml_ref, out_ref):
    gc = gc_ref[0]
    lc = lc_ref[0]
    q0 = q0_ref[0]
    q1 = q1_ref[0]
    mg = mg_ref[0].astype(jnp.float32)
    ml = ml_ref[0].astype(jnp.float32)

    ones_row = jnp.ones((1, _C), jnp.float32)
    zrow = jnp.zeros((1, _C), jnp.float32)

    q1n = q1 * _row_inv_norm(q1)
    lc_s, v8, _, _ = _side(lc, q1n, zrow, ml, ones_row)
    b8 = jax.lax.dot_general(v8, lc, _CONTRACT_N,
                             preferred_element_type=jnp.float32)
    lc_sum = b8[_Q:_Q + 1]

    q0n = q0 * _row_inv_norm(q0)
    gc_s, w8, inv_g, rm_raw = _side(gc, q0n, lc_sum, mg, ones_row)
    a8 = jax.lax.dot_general(w8, gc, _CONTRACT_N,
                             preferred_element_type=jnp.float32)

    rowmean = rm_raw * inv_g * (1.0 / _N)
    s_lc = jnp.sum(lc_s * ml, axis=1, keepdims=True)
    tvec = jnp.sum(gc_s * s_lc, axis=0, keepdims=True) * mg

    ab = a8 * b8
    row8 = jax.lax.broadcasted_iota(jnp.int32, (8, _C), 0)
    p1 = jnp.sum(jnp.where(row8 < _Q, ab, 0.0))
    g = jnp.sum(jnp.where(row8 == _Q, ab, 0.0))
    p2 = jnp.sum(tvec)
    p3 = jnp.sum(tvec * rowmean)

    z = jnp.concatenate([q0n, q1n], axis=0)
    sim = jax.lax.dot_general(z, z, _CONTRACT_C,
                              preferred_element_type=jnp.float32)
    ri = jax.lax.broadcasted_iota(jnp.int32, (2 * _Q, 2 * _Q), 0)
    ci = jax.lax.broadcasted_iota(jnp.int32, (2 * _Q, 2 * _Q), 1)
    simm = jnp.where(ri == ci, -_BIG_NEG, sim)
    m = jnp.max(simm, axis=1, keepdims=True)
    lse = jnp.log(jnp.sum(jnp.exp(simm - m), axis=1, keepdims=True)) + m
    pos = jnp.sum(jnp.where(ci == (ri + _Q) % (2 * _Q), sim, 0.0),
                  axis=1, keepdims=True)
    ce_sum = jnp.sum(lse - pos)

    lane = jax.lax.broadcasted_iota(jnp.int32, (1, 1, 128), 2)
    out_ref[...] = (jnp.where(lane == 0, p1, 0.0)
                    + jnp.where(lane == 1, p2, 0.0)
                    + jnp.where(lane == 2, p3, 0.0)
                    + jnp.where(lane == 3, g, 0.0)
                    + jnp.where(lane == 4, ce_sum, 0.0))


def _combine_kernel(p_ref, out_ref):
    p = p_ref[...]
    s = jnp.sum(p, axis=0, keepdims=True)
    lane = jax.lax.broadcasted_iota(jnp.int32, (1, 128), 1)

    def pick(i):
        return jnp.sum(jnp.where(lane == i, s, 0.0))

    s1, s2, s3, sg, sce = pick(0), pick(1), pick(2), pick(3), pick(4)
    old_mean = sg / (_B * _N * _N)
    cor_loss = -0.15 * (s1 + (old_mean - _NEG_PRESSURE) * s2 - s3)
    qa_loss = sce / (_B * 2 * _Q)
    out_ref[...] = jnp.where(lane == 0, cor_loss + qa_loss, 0.0)


def kernel(all_queries_0, all_queries_1, gc_output, lc_output,
           attn_hard, gc_spatial_res, lc_spatial_res):
    del gc_spatial_res, lc_spatial_res
    lc = lc_output[:, 0]
    attn3 = attn_hard.reshape(2 * _B, 1, _N)

    partials = pl.pallas_call(
        _batch_kernel,
        grid=(_B,),
        in_specs=[
            pl.BlockSpec((1, _Q, _C), lambda b: (b, 0, 0)),
            pl.BlockSpec((1, _Q, _C), lambda b: (b, 0, 0)),
            pl.BlockSpec((1, _N, _C), lambda b: (b, 0, 0)),
            pl.BlockSpec((1, _N, _C), lambda b: (b, 0, 0)),
            pl.BlockSpec((1, 1, _N), lambda b: (b, 0, 0)),
            pl.BlockSpec((1, 1, _N), lambda b: (b + _B, 0, 0)),
        ],
        out_specs=pl.BlockSpec((1, 1, 128), lambda b: (b, 0, 0)),
        out_shape=jax.ShapeDtypeStruct((_B, 1, 128), jnp.float32),
        compiler_params=pltpu.CompilerParams(
            dimension_semantics=("parallel",)),
    )(all_queries_0, all_queries_1, gc_output, lc, attn3, attn3)

    out = pl.pallas_call(
        _combine_kernel,
        grid=(1,),
        in_specs=[pl.BlockSpec((_B, 128), lambda i: (0, 0))],
        out_specs=pl.BlockSpec((1, 128), lambda i: (0, 0)),
        out_shape=jax.ShapeDtypeStruct((1, 128), jnp.float32),
    )(partials.reshape(_B, 128))
    return out[0, 0]


# speedup vs baseline: 5.0491x; 1.0130x over previous
import jax
import jax.numpy as jnp
from jax.experimental import pallas as pl
from jax.experimental.pallas import tpu as pltpu

_B, _RES, _C, _Q = 64, 28, 384, 5
_N = _RES * _RES
_NEG_PRESSURE = 0.1
_BIG_NEG = 1e30

_CONTRACT_C = (((1,), (1,)), ((), ()))
_CONTRACT_N = (((1,), (0,)), ((), ()))


def _row_inv_norm(x):
    ss = jnp.sum(x * x, axis=-1, keepdims=True)
    return 1.0 / jnp.maximum(jnp.sqrt(ss), 1e-10)


def _fold128(x):
    l = x.shape[1]
    acc = x[:, :128]
    for i in range(1, l // 128):
        acc = acc + x[:, 128 * i:128 * (i + 1)]
    rem = l % 128
    if rem:
        tail = jnp.concatenate(
            [x[:, l - rem:], jnp.zeros((1, 128 - rem), x.dtype)], axis=1)
        acc = acc + tail
    return acc


def _softmax_weights(data, qn, mask_row, ones_bf):
    sq = data * data
    ssq = jax.lax.dot_general(ones_bf, sq.astype(jnp.bfloat16), _CONTRACT_C,
                              preferred_element_type=jnp.float32)
    inv = 1.0 / jnp.maximum(jnp.sqrt(ssq), 1e-10)
    raw = jax.lax.dot_general(qn, data, _CONTRACT_C,
                              preferred_element_type=jnp.float32)
    a = jnp.maximum(raw * inv, 0.0)
    e = jnp.exp(a - jnp.max(a, axis=0, keepdims=True))
    soft = e / jnp.sum(e, axis=0, keepdims=True)
    return inv, soft


def _batch_kernel(q0_ref, q1_ref, gc_ref, lc_ref, mg_ref, ml_ref, out_ref):
    gc = gc_ref[0]
    lc = lc_ref[0]
    q0 = q0_ref[0]
    q1 = q1_ref[0]
    mg = mg_ref[0].astype(jnp.float32)
    ml = ml_ref[0].astype(jnp.float32)

    ones_bf = jnp.ones((1, _C), jnp.bfloat16)
    zrow_n = jnp.zeros((1, _N), jnp.float32)

    q0n = q0 * _row_inv_norm(q0)
    q1n = q1 * _row_inv_norm(q1)

    inv_l, lc_s = _softmax_weights(lc, q1n, ml, ones_bf)
    inv_g, gc_s = _softmax_weights(gc, q0n, mg, ones_bf)

    s_lc = jnp.sum(lc_s * ml, axis=1, keepdims=True)
    tvec = jnp.sum(gc_s * s_lc, axis=0, keepdims=True) * mg

    w8 = jnp.concatenate(
        [gc_s * mg * inv_g, inv_g, tvec * inv_g, zrow_n], axis=0)
    v8 = jnp.concatenate(
        [lc_s * ml * inv_l, inv_l, inv_l, zrow_n], axis=0)
    a8 = jax.lax.dot_general(w8, gc, _CONTRACT_N,
                             preferred_element_type=jnp.float32)
    b8 = jax.lax.dot_general(v8, lc, _CONTRACT_N,
                             preferred_element_type=jnp.float32)
    ab = a8 * b8
    p1_l = _fold128(jnp.sum(ab[:_Q], axis=0, keepdims=True))
    g_l = _fold128(ab[_Q:_Q + 1])
    p3_l = _fold128(ab[_Q + 1:_Q + 2])
    p2_l = _fold128(tvec)

    z = jnp.concatenate([q0n, q1n], axis=0)
    sim = jax.lax.dot_general(z, z, _CONTRACT_C,
                              preferred_element_type=jnp.float32)
    ri = jax.lax.broadcasted_iota(jnp.int32, (2 * _Q, 2 * _Q), 0)
    ci = jax.lax.broadcasted_iota(jnp.int32, (2 * _Q, 2 * _Q), 1)
    simm = jnp.where(ri == ci, -_BIG_NEG, sim)
    m = jnp.max(simm, axis=1, keepdims=True)
    lse = jnp.log(jnp.sum(jnp.exp(simm - m), axis=1, keepdims=True)) + m
    pos = jnp.sum(jnp.where(ci == (ri + _Q) % (2 * _Q), sim, 0.0),
                  axis=1, keepdims=True)
    ce_sum = jnp.sum(lse - pos)
    lane = jax.lax.broadcasted_iota(jnp.int32, (1, 128), 1)
    ce_l = jnp.where(lane == 0, ce_sum, 0.0)

    rows = jnp.concatenate(
        [p1_l, g_l, p3_l, p2_l, ce_l, jnp.zeros((3, 128), jnp.float32)],
        axis=0)
    out_ref[...] = rows.reshape(1, 8, 128)


def _combine_kernel(p_ref, out_ref):
    p = p_ref[...]
    s = jnp.sum(p, axis=0)

    def pick(i):
        return jnp.sum(s[i:i + 1])

    s1, sg, s3, s2, sce = pick(0), pick(1), pick(2), pick(3), pick(4)
    old_mean = sg / (_B * _N * _N)
    cor_loss = -0.15 * (s1 + (old_mean - _NEG_PRESSURE) * s2 - s3 / _N)
    qa_loss = sce / (_B * 2 * _Q)
    lane = jax.lax.broadcasted_iota(jnp.int32, (1, 128), 1)
    out_ref[...] = jnp.where(lane == 0, cor_loss + qa_loss, 0.0)


def kernel(all_queries_0, all_queries_1, gc_output, lc_output,
           attn_hard, gc_spatial_res, lc_spatial_res):
    del gc_spatial_res, lc_spatial_res
    lc = lc_output[:, 0]
    attn3 = attn_hard.reshape(2 * _B, 1, _N)

    partials = pl.pallas_call(
        _batch_kernel,
        grid=(_B,),
        in_specs=[
            pl.BlockSpec((1, _Q, _C), lambda b: (b, 0, 0)),
            pl.BlockSpec((1, _Q, _C), lambda b: (b, 0, 0)),
            pl.BlockSpec((1, _N, _C), lambda b: (b, 0, 0)),
            pl.BlockSpec((1, _N, _C), lambda b: (b, 0, 0)),
            pl.BlockSpec((1, 1, _N), lambda b: (b, 0, 0)),
            pl.BlockSpec((1, 1, _N), lambda b: (b + _B, 0, 0)),
        ],
        out_specs=pl.BlockSpec((1, 8, 128), lambda b: (b, 0, 0)),
        out_shape=jax.ShapeDtypeStruct((_B, 8, 128), jnp.float32),
        compiler_params=pltpu.CompilerParams(
            dimension_semantics=("arbitrary",)),
    )(all_queries_0, all_queries_1, gc_output, lc, attn3, attn3)

    out = pl.pallas_call(
        _combine_kernel,
        grid=(1,),
        in_specs=[pl.BlockSpec((_B, 8, 128), lambda i: (0, 0, 0))],
        out_specs=pl.BlockSpec((1, 128), lambda i: (0, 0)),
        out_shape=jax.ShapeDtypeStruct((1, 128), jnp.float32),
    )(partials)
    return out[0, 0]


# speedup vs baseline: 6.8379x; 1.3543x over previous
import jax
import jax.numpy as jnp
from jax.experimental import pallas as pl
from jax.experimental.pallas import tpu as pltpu

_B, _RES, _C, _Q = 64, 28, 384, 5
_N = _RES * _RES
_KB = 4
_NEG_PRESSURE = 0.1
_BIG_NEG = 1e30

_CONTRACT_C = (((1,), (1,)), ((), ()))
_CONTRACT_N = (((1,), (0,)), ((), ()))


def _row_inv_norm(x):
    ss = jnp.sum(x * x, axis=-1, keepdims=True)
    return 1.0 / jnp.maximum(jnp.sqrt(ss), 1e-10)


def _fold128(x):
    l = x.shape[1]
    acc = x[:, :128]
    for i in range(1, l // 128):
        acc = acc + x[:, 128 * i:128 * (i + 1)]
    rem = l % 128
    if rem:
        tail = jnp.concatenate(
            [x[:, l - rem:], jnp.zeros((1, 128 - rem), x.dtype)], axis=1)
        acc = acc + tail
    return acc


def _softmax_weights(d_bf, qn, ones_bf):
    ssq = jax.lax.dot_general(ones_bf, d_bf * d_bf, _CONTRACT_C,
                              preferred_element_type=jnp.float32)
    inv = 1.0 / jnp.maximum(jnp.sqrt(ssq), 1e-10)
    raw = jax.lax.dot_general(qn.astype(jnp.bfloat16), d_bf, _CONTRACT_C,
                              preferred_element_type=jnp.float32)
    a = jnp.maximum(raw * inv, 0.0)
    e = jnp.exp(a - jnp.max(a, axis=0, keepdims=True))
    soft = e / jnp.sum(e, axis=0, keepdims=True)
    return inv, soft


def _one_batch(gc, lc, q0, q1, mg, ml, ones_bf, zrow_n, lane):
    q0n = q0 * _row_inv_norm(q0)
    q1n = q1 * _row_inv_norm(q1)

    gc_bf = gc.astype(jnp.bfloat16)
    lc_bf = lc.astype(jnp.bfloat16)
    inv_l, lc_s = _softmax_weights(lc_bf, q1n, ones_bf)
    inv_g, gc_s = _softmax_weights(gc_bf, q0n, ones_bf)

    s_lc = jnp.sum(lc_s * ml, axis=1, keepdims=True)
    tvec = jnp.sum(gc_s * s_lc, axis=0, keepdims=True) * mg

    w8 = jnp.concatenate(
        [gc_s * mg * inv_g, inv_g, tvec * inv_g, zrow_n], axis=0)
    v8 = jnp.concatenate(
        [lc_s * ml * inv_l, inv_l, inv_l, zrow_n], axis=0)
    a8 = jax.lax.dot_general(w8.astype(jnp.bfloat16), gc_bf, _CONTRACT_N,
                             preferred_element_type=jnp.float32)
    b8 = jax.lax.dot_general(v8.astype(jnp.bfloat16), lc_bf, _CONTRACT_N,
                             preferred_element_type=jnp.float32)
    ab = a8 * b8
    p1_l = _fold128(jnp.sum(ab[:_Q], axis=0, keepdims=True))
    g_l = _fold128(ab[_Q:_Q + 1])
    p3_l = _fold128(ab[_Q + 1:_Q + 2])
    p2_l = _fold128(tvec)

    z = jnp.concatenate([q0n, q1n], axis=0)
    sim = jax.lax.dot_general(z, z, _CONTRACT_C,
                              preferred_element_type=jnp.float32)
    ri = jax.lax.broadcasted_iota(jnp.int32, (2 * _Q, 2 * _Q), 0)
    ci = jax.lax.broadcasted_iota(jnp.int32, (2 * _Q, 2 * _Q), 1)
    simm = jnp.where(ri == ci, -_BIG_NEG, sim)
    m = jnp.max(simm, axis=1, keepdims=True)
    lse = jnp.log(jnp.sum(jnp.exp(simm - m), axis=1, keepdims=True)) + m
    pos = jnp.sum(jnp.where(ci == (ri + _Q) % (2 * _Q), sim, 0.0),
                  axis=1, keepdims=True)
    ce_sum = jnp.sum(lse - pos)
    ce_l = jnp.where(lane == 0, ce_sum, 0.0)

    return jnp.concatenate(
        [p1_l, g_l, p3_l, p2_l, ce_l, jnp.zeros((3, 128), jnp.float32)],
        axis=0)


def _batch_kernel(q0_ref, q1_ref, gc_ref, lc_ref, mg_ref, ml_ref, out_ref):
    ones_bf = jnp.ones((1, _C), jnp.bfloat16)
    zrow_n = jnp.zeros((1, _N), jnp.float32)
    lane = jax.lax.broadcasted_iota(jnp.int32, (1, 128), 1)

    rows = jnp.zeros((8, 128), jnp.float32)
    for i in range(_KB):
        rows = rows + _one_batch(
            gc_ref[i], lc_ref[i], q0_ref[i], q1_ref[i],
            mg_ref[i].astype(jnp.float32), ml_ref[i].astype(jnp.float32),
            ones_bf, zrow_n, lane)
    out_ref[...] = rows.reshape(1, 8, 128)


def _combine_kernel(p_ref, out_ref):
    p = p_ref[...]
    s = jnp.sum(p, axis=0)

    def pick(i):
        return jnp.sum(s[i:i + 1])

    s1, sg, s3, s2, sce = pick(0), pick(1), pick(2), pick(3), pick(4)
    old_mean = sg / (_B * _N * _N)
    cor_loss = -0.15 * (s1 + (old_mean - _NEG_PRESSURE) * s2 - s3 / _N)
    qa_loss = sce / (_B * 2 * _Q)
    lane = jax.lax.broadcasted_iota(jnp.int32, (1, 128), 1)
    out_ref[...] = jnp.where(lane == 0, cor_loss + qa_loss, 0.0)


def kernel(all_queries_0, all_queries_1, gc_output, lc_output,
           attn_hard, gc_spatial_res, lc_spatial_res):
    del gc_spatial_res, lc_spatial_res
    lc = lc_output[:, 0]
    attn3 = attn_hard.reshape(2 * _B, 1, _N)
    steps = _B // _KB

    partials = pl.pallas_call(
        _batch_kernel,
        grid=(steps,),
        in_specs=[
            pl.BlockSpec((_KB, _Q, _C), lambda b: (b, 0, 0)),
            pl.BlockSpec((_KB, _Q, _C), lambda b: (b, 0, 0)),
            pl.BlockSpec((_KB, _N, _C), lambda b: (b, 0, 0)),
            pl.BlockSpec((_KB, _N, _C), lambda b: (b, 0, 0)),
            pl.BlockSpec((_KB, 1, _N), lambda b: (b, 0, 0)),
            pl.BlockSpec((_KB, 1, _N), lambda b: (b + steps, 0, 0)),
        ],
        out_specs=pl.BlockSpec((1, 8, 128), lambda b: (b, 0, 0)),
        out_shape=jax.ShapeDtypeStruct((steps, 8, 128), jnp.float32),
        compiler_params=pltpu.CompilerParams(
            dimension_semantics=("arbitrary",)),
    )(all_queries_0, all_queries_1, gc_output, lc, attn3, attn3)

    out = pl.pallas_call(
        _combine_kernel,
        grid=(1,),
        in_specs=[pl.BlockSpec((steps, 8, 128), lambda i: (0, 0, 0))],
        out_specs=pl.BlockSpec((1, 128), lambda i: (0, 0)),
        out_shape=jax.ShapeDtypeStruct((1, 128), jnp.float32),
    )(partials)
    return out[0, 0]


# speedup vs baseline: 6.9441x; 1.0155x over previous
import jax
import jax.numpy as jnp
from jax.experimental import pallas as pl
from jax.experimental.pallas import tpu as pltpu

_B, _RES, _C, _Q = 64, 28, 384, 5
_N = _RES * _RES
_KB = 8
_NEG_PRESSURE = 0.1
_BIG_NEG = 1e30

_CONTRACT_C = (((1,), (1,)), ((), ()))
_CONTRACT_N = (((1,), (0,)), ((), ()))


def _row_inv_norm(x):
    ss = jnp.sum(x * x, axis=-1, keepdims=True)
    return 1.0 / jnp.maximum(jnp.sqrt(ss), 1e-10)


def _fold128(x):
    l = x.shape[1]
    acc = x[:, :128]
    for i in range(1, l // 128):
        acc = acc + x[:, 128 * i:128 * (i + 1)]
    rem = l % 128
    if rem:
        tail = jnp.concatenate(
            [x[:, l - rem:], jnp.zeros((1, 128 - rem), x.dtype)], axis=1)
        acc = acc + tail
    return acc


def _softmax_weights(d_bf, qn, ones_bf):
    ssq = jax.lax.dot_general(ones_bf, d_bf * d_bf, _CONTRACT_C,
                              preferred_element_type=jnp.float32)
    inv = 1.0 / jnp.maximum(jnp.sqrt(ssq), 1e-10)
    raw = jax.lax.dot_general(qn.astype(jnp.bfloat16), d_bf, _CONTRACT_C,
                              preferred_element_type=jnp.float32)
    a = jnp.maximum(raw * inv, 0.0)
    e = jnp.exp(a - jnp.max(a, axis=0, keepdims=True))
    soft = e / jnp.sum(e, axis=0, keepdims=True)
    return inv, soft


def _one_batch(gc, lc, q0, q1, mg, ml, ones_bf, zrow_n, lane):
    q0n = q0 * _row_inv_norm(q0)
    q1n = q1 * _row_inv_norm(q1)

    gc_bf = gc.astype(jnp.bfloat16)
    lc_bf = lc.astype(jnp.bfloat16)
    inv_l, lc_s = _softmax_weights(lc_bf, q1n, ones_bf)
    inv_g, gc_s = _softmax_weights(gc_bf, q0n, ones_bf)

    s_lc = jnp.sum(lc_s * ml, axis=1, keepdims=True)
    tvec = jnp.sum(gc_s * s_lc, axis=0, keepdims=True) * mg

    w8 = jnp.concatenate(
        [gc_s * mg * inv_g, inv_g, tvec * inv_g, zrow_n], axis=0)
    v8 = jnp.concatenate(
        [lc_s * ml * inv_l, inv_l, inv_l, zrow_n], axis=0)
    a8 = jax.lax.dot_general(w8.astype(jnp.bfloat16), gc_bf, _CONTRACT_N,
                             preferred_element_type=jnp.float32)
    b8 = jax.lax.dot_general(v8.astype(jnp.bfloat16), lc_bf, _CONTRACT_N,
                             preferred_element_type=jnp.float32)
    ab = a8 * b8
    p1_l = _fold128(jnp.sum(ab[:_Q], axis=0, keepdims=True))
    g_l = _fold128(ab[_Q:_Q + 1])
    p3_l = _fold128(ab[_Q + 1:_Q + 2])
    p2_l = _fold128(tvec)

    z = jnp.concatenate([q0n, q1n], axis=0)
    sim = jax.lax.dot_general(z, z, _CONTRACT_C,
                              preferred_element_type=jnp.float32)
    ri = jax.lax.broadcasted_iota(jnp.int32, (2 * _Q, 2 * _Q), 0)
    ci = jax.lax.broadcasted_iota(jnp.int32, (2 * _Q, 2 * _Q), 1)
    simm = jnp.where(ri == ci, -_BIG_NEG, sim)
    m = jnp.max(simm, axis=1, keepdims=True)
    lse = jnp.log(jnp.sum(jnp.exp(simm - m), axis=1, keepdims=True)) + m
    pos = jnp.sum(jnp.where(ci == (ri + _Q) % (2 * _Q), sim, 0.0),
                  axis=1, keepdims=True)
    ce_sum = jnp.sum(lse - pos)
    ce_l = jnp.where(lane == 0, ce_sum, 0.0)

    return jnp.concatenate(
        [p1_l, g_l, p3_l, p2_l, ce_l, jnp.zeros((3, 128), jnp.float32)],
        axis=0)


def _batch_kernel(q0_ref, q1_ref, gc_ref, lc_ref, mg_ref, ml_ref, out_ref):
    ones_bf = jnp.ones((1, _C), jnp.bfloat16)
    zrow_n = jnp.zeros((1, _N), jnp.float32)
    lane = jax.lax.broadcasted_iota(jnp.int32, (1, 128), 1)

    rows = jnp.zeros((8, 128), jnp.float32)
    for i in range(_KB):
        rows = rows + _one_batch(
            gc_ref[i], lc_ref[i], q0_ref[i], q1_ref[i],
            mg_ref[i].astype(jnp.float32), ml_ref[i].astype(jnp.float32),
            ones_bf, zrow_n, lane)
    out_ref[...] = rows.reshape(1, 8, 128)


def _combine_kernel(p_ref, out_ref):
    p = p_ref[...]
    s = jnp.sum(p, axis=0)

    def pick(i):
        return jnp.sum(s[i:i + 1])

    s1, sg, s3, s2, sce = pick(0), pick(1), pick(2), pick(3), pick(4)
    old_mean = sg / (_B * _N * _N)
    cor_loss = -0.15 * (s1 + (old_mean - _NEG_PRESSURE) * s2 - s3 / _N)
    qa_loss = sce / (_B * 2 * _Q)
    lane = jax.lax.broadcasted_iota(jnp.int32, (1, 128), 1)
    out_ref[...] = jnp.where(lane == 0, cor_loss + qa_loss, 0.0)


def kernel(all_queries_0, all_queries_1, gc_output, lc_output,
           attn_hard, gc_spatial_res, lc_spatial_res):
    del gc_spatial_res, lc_spatial_res
    lc = lc_output[:, 0]
    attn3 = attn_hard.reshape(2 * _B, 1, _N)
    steps = _B // _KB

    partials = pl.pallas_call(
        _batch_kernel,
        grid=(steps,),
        in_specs=[
            pl.BlockSpec((_KB, _Q, _C), lambda b: (b, 0, 0)),
            pl.BlockSpec((_KB, _Q, _C), lambda b: (b, 0, 0)),
            pl.BlockSpec((_KB, _N, _C), lambda b: (b, 0, 0)),
            pl.BlockSpec((_KB, _N, _C), lambda b: (b, 0, 0)),
            pl.BlockSpec((_KB, 1, _N), lambda b: (b, 0, 0)),
            pl.BlockSpec((_KB, 1, _N), lambda b: (b + steps, 0, 0)),
        ],
        out_specs=pl.BlockSpec((1, 8, 128), lambda b: (b, 0, 0)),
        out_shape=jax.ShapeDtypeStruct((steps, 8, 128), jnp.float32),
        compiler_params=pltpu.CompilerParams(
            dimension_semantics=("arbitrary",),
            vmem_limit_bytes=56 * 1024 * 1024),
    )(all_queries_0, all_queries_1, gc_output, lc, attn3, attn3)

    out = pl.pallas_call(
        _combine_kernel,
        grid=(1,),
        in_specs=[pl.BlockSpec((steps, 8, 128), lambda i: (0, 0, 0))],
        out_specs=pl.BlockSpec((1, 128), lambda i: (0, 0)),
        out_shape=jax.ShapeDtypeStruct((1, 128), jnp.float32),
    )(partials)
    return out[0, 0]


# speedup vs baseline: 7.0935x; 1.0215x over previous
import jax
import jax.numpy as jnp
from jax.experimental import pallas as pl
from jax.experimental.pallas import tpu as pltpu

_B, _RES, _C, _Q = 64, 28, 384, 5
_N = _RES * _RES
_KB = 8
_NEG_PRESSURE = 0.1
_BIG_NEG = 1e30

_CONTRACT_C = (((1,), (1,)), ((), ()))
_CONTRACT_N = (((1,), (0,)), ((), ()))
_FAST = jax.lax.Precision.DEFAULT


def _row_inv_norm(x):
    ss = jnp.sum(x * x, axis=-1, keepdims=True)
    return 1.0 / jnp.maximum(jnp.sqrt(ss), 1e-10)


def _fold128(x):
    l = x.shape[1]
    acc = x[:, :128]
    for i in range(1, l // 128):
        acc = acc + x[:, 128 * i:128 * (i + 1)]
    rem = l % 128
    if rem:
        tail = jnp.concatenate(
            [x[:, l - rem:], jnp.zeros((1, 128 - rem), x.dtype)], axis=1)
        acc = acc + tail
    return acc


def _softmax_weights(d, qn, ones_row):
    ssq = jax.lax.dot_general(ones_row, d * d, _CONTRACT_C,
                              precision=_FAST,
                              preferred_element_type=jnp.float32)
    inv = jax.lax.rsqrt(jnp.maximum(ssq, 1e-20))
    raw = jax.lax.dot_general(qn, d, _CONTRACT_C, precision=_FAST,
                              preferred_element_type=jnp.float32)
    a = jnp.maximum(raw * inv, 0.0)
    e = jnp.exp(a - jnp.max(a, axis=0, keepdims=True))
    soft = e / jnp.sum(e, axis=0, keepdims=True)
    return inv, soft


def _one_batch(gc, lc, q0, q1, mg, ml, ones_row, zrow_n, lane):
    q0n = q0 * _row_inv_norm(q0)
    q1n = q1 * _row_inv_norm(q1)

    inv_l, lc_s = _softmax_weights(lc, q1n, ones_row)
    inv_g, gc_s = _softmax_weights(gc, q0n, ones_row)

    s_lc = jnp.sum(lc_s * ml, axis=1, keepdims=True)
    tvec = jnp.sum(gc_s * s_lc, axis=0, keepdims=True) * mg

    w8 = jnp.concatenate(
        [gc_s * mg * inv_g, inv_g, tvec * inv_g, zrow_n], axis=0)
    v8 = jnp.concatenate(
        [lc_s * ml * inv_l, inv_l, inv_l, zrow_n], axis=0)
    a8 = jax.lax.dot_general(w8, gc, _CONTRACT_N, precision=_FAST,
                             preferred_element_type=jnp.float32)
    b8 = jax.lax.dot_general(v8, lc, _CONTRACT_N, precision=_FAST,
                             preferred_element_type=jnp.float32)
    ab = a8 * b8
    p1_l = _fold128(jnp.sum(ab[:_Q], axis=0, keepdims=True))
    g_l = _fold128(ab[_Q:_Q + 1])
    p3_l = _fold128(ab[_Q + 1:_Q + 2])
    p2_l = _fold128(tvec)

    rows = jnp.concatenate(
        [p1_l, g_l, p3_l, p2_l, jnp.zeros((4, 128), jnp.float32)],
        axis=0)
    return rows, jnp.concatenate([q0n, q1n], axis=0)


def _batch_kernel(q0_ref, q1_ref, gc_ref, lc_ref, mg_ref, ml_ref, out_ref):
    ones_row = jnp.ones((1, _C), jnp.float32)
    zrow_n = jnp.zeros((1, _N), jnp.float32)
    lane = jax.lax.broadcasted_iota(jnp.int32, (1, 128), 1)

    rows = jnp.zeros((8, 128), jnp.float32)
    zs = []
    for i in range(_KB):
        r, z = _one_batch(
            gc_ref[i], lc_ref[i], q0_ref[i], q1_ref[i],
            mg_ref[i].astype(jnp.float32), ml_ref[i].astype(jnp.float32),
            ones_row, zrow_n, lane)
        rows = rows + r
        zs.append(z)

    t = 2 * _Q
    z_all = jnp.concatenate(zs, axis=0)
    sim = jax.lax.dot_general(z_all, z_all, _CONTRACT_C, precision=_FAST,
                              preferred_element_type=jnp.float32)
    ri = jax.lax.broadcasted_iota(jnp.int32, (_KB * t, _KB * t), 0)
    ci = jax.lax.broadcasted_iota(jnp.int32, (_KB * t, _KB * t), 1)
    off_diag_block = (ri != ci) & (ri // t == ci // t)
    simm = jnp.where(off_diag_block, sim, -_BIG_NEG)
    m = jnp.max(simm, axis=1, keepdims=True)
    lse = jnp.log(jnp.sum(jnp.exp(simm - m), axis=1, keepdims=True)) + m
    pos_mask = ci == (ri // t) * t + (ri % t + _Q) % t
    pos = jnp.sum(jnp.where(pos_mask, sim, 0.0), axis=1, keepdims=True)
    ce_sum = jnp.sum(lse - pos)
    ce_l = jnp.where(lane == 0, ce_sum, 0.0)

    rows = rows + jnp.concatenate(
        [jnp.zeros((4, 128), jnp.float32), ce_l,
         jnp.zeros((3, 128), jnp.float32)], axis=0)
    out_ref[...] = rows.reshape(1, 8, 128)


def _combine_kernel(p_ref, out_ref):
    p = p_ref[...]
    s = jnp.sum(p, axis=0)

    def pick(i):
        return jnp.sum(s[i:i + 1])

    s1, sg, s3, s2, sce = pick(0), pick(1), pick(2), pick(3), pick(4)
    old_mean = sg / (_B * _N * _N)
    cor_loss = -0.15 * (s1 + (old_mean - _NEG_PRESSURE) * s2 - s3 / _N)
    qa_loss = sce / (_B * 2 * _Q)
    lane = jax.lax.broadcasted_iota(jnp.int32, (1, 128), 1)
    out_ref[...] = jnp.where(lane == 0, cor_loss + qa_loss, 0.0)


def kernel(all_queries_0, all_queries_1, gc_output, lc_output,
           attn_hard, gc_spatial_res, lc_spatial_res):
    del gc_spatial_res, lc_spatial_res
    lc = lc_output[:, 0]
    attn3 = attn_hard.reshape(2 * _B, 1, _N)
    steps = _B // _KB

    partials = pl.pallas_call(
        _batch_kernel,
        grid=(steps,),
        in_specs=[
            pl.BlockSpec((_KB, _Q, _C), lambda b: (b, 0, 0)),
            pl.BlockSpec((_KB, _Q, _C), lambda b: (b, 0, 0)),
            pl.BlockSpec((_KB, _N, _C), lambda b: (b, 0, 0)),
            pl.BlockSpec((_KB, _N, _C), lambda b: (b, 0, 0)),
            pl.BlockSpec((_KB, 1, _N), lambda b: (b, 0, 0)),
            pl.BlockSpec((_KB, 1, _N), lambda b: (b + steps, 0, 0)),
        ],
        out_specs=pl.BlockSpec((1, 8, 128), lambda b: (b, 0, 0)),
        out_shape=jax.ShapeDtypeStruct((steps, 8, 128), jnp.float32),
        compiler_params=pltpu.CompilerParams(
            dimension_semantics=("arbitrary",),
            vmem_limit_bytes=56 * 1024 * 1024),
    )(all_queries_0, all_queries_1, gc_output, lc, attn3, attn3)

    out = pl.pallas_call(
        _combine_kernel,
        grid=(1,),
        in_specs=[pl.BlockSpec((steps, 8, 128), lambda i: (0, 0, 0))],
        out_specs=pl.BlockSpec((1, 128), lambda i: (0, 0)),
        out_shape=jax.ShapeDtypeStruct((1, 128), jnp.float32),
    )(partials)
    return out[0, 0]
